# Initial kernel scaffold; baseline (speedup 1.0000x reference)
import jax
import jax.numpy as jnp
from jax.experimental import pallas as pl


def kernel(at_no, coord, edge_index, shifts, charge, spin, params):
    raise NotImplementedError("write your pallas kernel here")


# speedup vs baseline: 1.9652x; 1.9652x over previous
import functools

import jax
import jax.numpy as jnp
from jax import lax
from jax.experimental import pallas as pl
from jax.experimental.pallas import tpu as pltpu
from jax.experimental.pallas import tpu_sc as plsc

NODE_DIM = 128
C_TOT = 224
V_DIM = 480
NUM_BASIS = 20
CUTOFF = 5.0
N_BLOCKS = 3
N_NODES = 10000
N_EDGES = 320000


_CH = 400
_NCHUNK_W = 25
_NPAD = 10240
_ACCW = _NPAD * 12


def _fv_body(gf, vf, src_h, dst_h, out, g_v, v_v, si_v, di_v, acc):
    c = lax.axis_index("c")
    s = lax.axis_index("s")
    w = s * 2 + c

    cvec = lax.iota(jnp.int32, 16)
    zeros = jnp.zeros((16,), jnp.float32)

    def zfill(i, carry):
        acc[pl.ds(i * 16, 16)] = zeros
        return carry

    lax.fori_loop(0, _ACCW // 16, zfill, 0)

    def do_chunk(i, carry):
        base = w * (_CH * _NCHUNK_W) + i * _CH
        pltpu.sync_copy(gf.at[pl.ds(base * 3, _CH * 3)], g_v)
        pltpu.sync_copy(vf.at[pl.ds(base * 3, _CH * 3)], v_v)
        pltpu.sync_copy(src_h.at[pl.ds(base, _CH)], si_v)
        pltpu.sync_copy(dst_h.at[pl.ds(base, _CH)], di_v)

        def group(g, carry2):
            l0 = g * 16
            se12 = si_v[pl.ds(l0, 16)] * 12
            de12 = di_v[pl.ds(l0, 16)] * 12
            tri = cvec * 3 + 3 * l0
            gc = [plsc.load_gather(g_v, [tri + a]) for a in range(3)]
            vc = [plsc.load_gather(v_v, [tri + b]) for b in range(3)]
            for a in range(3):
                plsc.addupdate_scatter(acc, [se12 + a], -gc[a])
                plsc.addupdate_scatter(acc, [de12 + a], gc[a])
                for b in range(3):
                    h = 0.5 * gc[a] * vc[b]
                    cc = 3 + 3 * a + b
                    plsc.addupdate_scatter(acc, [se12 + cc], h)
                    plsc.addupdate_scatter(acc, [de12 + cc], h)
            return carry2

        lax.fori_loop(0, _CH // 16, group, 0)
        return carry

    lax.fori_loop(0, _NCHUNK_W, do_chunk, 0)
    pltpu.sync_copy(acc, out.at[w])


_fv_kernel = functools.partial(
    pl.kernel,
    out_type=jax.ShapeDtypeStruct((32, _ACCW), jnp.float32),
    mesh=plsc.VectorSubcoreMesh(core_axis_name="c", subcore_axis_name="s"),
    compiler_params=pltpu.CompilerParams(needs_layout_passes=False),
    scratch_types=[
        pltpu.VMEM((3 * _CH,), jnp.float32),
        pltpu.VMEM((3 * _CH,), jnp.float32),
        pltpu.VMEM((_CH,), jnp.int32),
        pltpu.VMEM((_CH,), jnp.int32),
        pltpu.VMEM((_ACCW,), jnp.float32),
    ],
)(_fv_body)


def _force_virial(edge_grad, vec, src, dst):
    gf = edge_grad.reshape(-1)
    vf = vec.reshape(-1)
    acc = jnp.sum(_fv_kernel(gf, vf, src, dst), axis=0)
    return acc.reshape(_NPAD, 12)[:N_NODES]


def _silu(x):
    return x * jax.nn.sigmoid(x)


def _expand_gate(g):
    g0 = g[..., :128]
    g1 = jnp.repeat(g[..., 128:192], 3, axis=-1)
    g2 = jnp.repeat(g[..., 192:224], 5, axis=-1)
    return jnp.concatenate([g0, g1, g2], axis=-1)


def _sph_harm(u):
    x, y, z = u[:, 0], u[:, 1], u[:, 2]
    l0 = jnp.ones_like(x)[:, None]
    l1 = jnp.sqrt(3.0) * u
    l2 = jnp.stack([
        jnp.sqrt(15.0) * x * y,
        jnp.sqrt(15.0) * y * z,
        jnp.sqrt(5.0) / 2.0 * (2.0 * z * z - x * x - y * y),
        jnp.sqrt(15.0) * x * z,
        jnp.sqrt(15.0) / 2.0 * (x * x - y * y),
    ], axis=-1)
    return l0, l1, l2


def _tile_rsh(l0, l1, l2):
    E = l0.shape[0]
    r0 = jnp.broadcast_to(l0, (E, 128))
    r1 = jnp.broadcast_to(l1[:, None, :], (E, 64, 3)).reshape(E, 192)
    r2 = jnp.broadcast_to(l2[:, None, :], (E, 32, 5)).reshape(E, 160)
    return jnp.concatenate([r0, r1, r2], axis=-1)


def _bessel_rbf(dist):
    n = jnp.arange(1, NUM_BASIS + 1, dtype=jnp.float32)
    return jnp.sqrt(2.0 / CUTOFF) * jnp.sin(n * jnp.pi * dist / CUTOFF) / dist


def _cosine_cutoff(dist):
    return 0.5 * (jnp.cos(jnp.pi * dist / CUTOFF) + 1.0) * (dist < CUTOFF).astype(jnp.float32)


def kernel(at_no, coord, edge_index, shifts, charge, spin, params):
    src = edge_index[0]
    dst = edge_index[1]
    N = coord.shape[0]
    vec = coord[src] - coord[dst] - shifts
    x = params['embed_table'][at_no]
    x_scalar0 = x @ params['node_lin_W'] + params['node_lin_b']

    def energy_fn(vec):
        dist = jnp.sqrt(jnp.sum(vec * vec, axis=-1, keepdims=True) + 1e-10)
        rbf = _bessel_rbf(dist)
        fcut = _cosine_cutoff(dist)
        u = vec[:, jnp.array([1, 2, 0])] / dist
        l0, l1, l2 = _sph_harm(u)
        rsh = _tile_rsh(l0, l1, l2)
        xs = x_scalar0
        xv = jnp.zeros((N, V_DIM), jnp.float32)
        for b in range(N_BLOCKS):
            phi = _silu(xs @ params['msg_W1'][b] + params['msg_b1'][b])
            phi = phi @ params['msg_W2'][b] + params['msg_b2'][b]
            W = (rbf @ params['msg_Wrbf'][b] + params['msg_brbf'][b]) * fcut
            pr = phi[dst] * W
            ds = pr[:, :NODE_DIM]
            gv1 = pr[:, NODE_DIM:NODE_DIM + C_TOT]
            gv2 = pr[:, NODE_DIM + C_TOT:]
            dv = xv[dst] * _expand_gate(gv1) + rsh * _expand_gate(gv2)
            xs = xs + jax.ops.segment_sum(ds, src, num_segments=N)
            xv = xv + jax.ops.segment_sum(dv, src, num_segments=N)
            v0 = xv[:, :128]
            v1 = xv[:, 128:320].reshape(N, 64, 3)
            v2 = xv[:, 320:].reshape(N, 32, 5)
            U0 = v0 @ params['upd_U0'][b]
            V0 = v0 @ params['upd_V0'][b]
            U1 = jnp.einsum('ncd,ce->ned', v1, params['upd_U1'][b])
            V1 = jnp.einsum('ncd,ce->ned', v1, params['upd_V1'][b])
            U2 = jnp.einsum('ncd,ce->ned', v2, params['upd_U2'][b])
            V2 = jnp.einsum('ncd,ce->ned', v2, params['upd_V2'][b])
            norms = jnp.concatenate([
                jnp.sqrt(V0 * V0 + 1e-10),
                jnp.sqrt(jnp.sum(V1 * V1, axis=-1) + 1e-10),
                jnp.sqrt(jnp.sum(V2 * V2, axis=-1) + 1e-10)], axis=-1)
            h = _silu(jnp.concatenate([xs, norms], axis=-1) @ params['upd_W1'][b] + params['upd_b1'][b])
            h = h @ params['upd_W2'][b] + params['upd_b2'][b]
            a_ss = h[:, :NODE_DIM]
            a_sv = h[:, NODE_DIM:2 * NODE_DIM]
            a_vv = h[:, 2 * NODE_DIM:]
            dot = jnp.concatenate([U0 * V0, jnp.sum(U1 * V1, axis=-1), jnp.sum(U2 * V2, axis=-1)], axis=-1)
            xs = xs + a_ss + a_sv * (dot @ params['upd_Wdot'][b])
            Uflat = jnp.concatenate([U0, U1.reshape(N, 192), U2.reshape(N, 160)], axis=-1)
            xv = xv + _expand_gate(a_vv) * Uflat
        hh = _silu(xs @ params['out_W1'] + params['out_b1'])
        energies = (hh @ params['out_W2'] + params['out_b2']).reshape(-1)
        return jnp.sum(energies), energies

    (etot, energies_raw), edge_grad = jax.value_and_grad(energy_fn, has_aux=True)(vec)
    energies = energies_raw + params['atom_sp'][at_no]
    energy = jnp.sum(energies)

    acc = _force_virial(edge_grad, vec, src, dst)
    forces = acc[:, :3]
    virials = acc[:, 3:12].reshape(N, 3, 3)
    virial = jnp.sum(virials, axis=0)
    virial = 0.5 * (virial + virial.T)
    return energy, energies, forces, virial, virials


# speedup vs baseline: 2.3824x; 1.2123x over previous
import functools

import jax
import jax.numpy as jnp
from jax import lax
from jax.experimental import pallas as pl
from jax.experimental.pallas import tpu as pltpu
from jax.experimental.pallas import tpu_sc as plsc

NODE_DIM = 128
C_TOT = 224
V_DIM = 480
NUM_BASIS = 20
CUTOFF = 5.0
N_BLOCKS = 3
N_NODES = 10000
N_EDGES = 320000


_CH = 400
_NCHUNK_W = 25
_NPAD = 10240
_ACCW = _NPAD * 12


def _fv_body(gf, vf, src_h, dst_h, out, g_v, v_v, si_v, di_v, acc):
    c = lax.axis_index("c")
    s = lax.axis_index("s")
    w = s * 2 + c

    cvec = lax.iota(jnp.int32, 16)
    zeros = jnp.zeros((16,), jnp.float32)

    def zfill(i, carry):
        acc[pl.ds(i * 16, 16)] = zeros
        return carry

    lax.fori_loop(0, _ACCW // 16, zfill, 0)

    def do_chunk(i, carry):
        base = w * (_CH * _NCHUNK_W) + i * _CH
        pltpu.sync_copy(gf.at[pl.ds(base * 3, _CH * 3)], g_v)
        pltpu.sync_copy(vf.at[pl.ds(base * 3, _CH * 3)], v_v)
        pltpu.sync_copy(src_h.at[pl.ds(base, _CH)], si_v)
        pltpu.sync_copy(dst_h.at[pl.ds(base, _CH)], di_v)

        def group(g, carry2):
            l0 = g * 16
            se12 = si_v[pl.ds(l0, 16)] * 12
            de12 = di_v[pl.ds(l0, 16)] * 12
            tri = cvec * 3 + 3 * l0
            gc = [plsc.load_gather(g_v, [tri + a]) for a in range(3)]
            vc = [plsc.load_gather(v_v, [tri + b]) for b in range(3)]
            for a in range(3):
                plsc.addupdate_scatter(acc, [se12 + a], -gc[a])
                plsc.addupdate_scatter(acc, [de12 + a], gc[a])
                for b in range(3):
                    h = 0.5 * gc[a] * vc[b]
                    cc = 3 + 3 * a + b
                    plsc.addupdate_scatter(acc, [se12 + cc], h)
                    plsc.addupdate_scatter(acc, [de12 + cc], h)
            return carry2

        lax.fori_loop(0, _CH // 16, group, 0)
        return carry

    lax.fori_loop(0, _NCHUNK_W, do_chunk, 0)
    pltpu.sync_copy(acc, out.at[w])


_fv_kernel = functools.partial(
    pl.kernel,
    out_type=jax.ShapeDtypeStruct((32, _ACCW), jnp.float32),
    mesh=plsc.VectorSubcoreMesh(core_axis_name="c", subcore_axis_name="s"),
    compiler_params=pltpu.CompilerParams(needs_layout_passes=False),
    scratch_types=[
        pltpu.VMEM((3 * _CH,), jnp.float32),
        pltpu.VMEM((3 * _CH,), jnp.float32),
        pltpu.VMEM((_CH,), jnp.int32),
        pltpu.VMEM((_CH,), jnp.int32),
        pltpu.VMEM((_ACCW,), jnp.float32),
    ],
)(_fv_body)


def _force_virial(edge_grad, vec, src, dst):
    gf = edge_grad.reshape(-1)
    vf = vec.reshape(-1)
    acc = jnp.sum(_fv_kernel(gf, vf, src, dst), axis=0)
    return acc.reshape(_NPAD, 12)[:N_NODES]


def _silu(x):
    return x * jax.nn.sigmoid(x)


def _sph_harm(u):
    x, y, z = u[:, 0], u[:, 1], u[:, 2]
    l1 = jnp.sqrt(3.0) * u
    l2 = jnp.stack([
        jnp.sqrt(15.0) * x * y,
        jnp.sqrt(15.0) * y * z,
        jnp.sqrt(5.0) / 2.0 * (2.0 * z * z - x * x - y * y),
        jnp.sqrt(15.0) * x * z,
        jnp.sqrt(15.0) / 2.0 * (x * x - y * y),
    ], axis=-1)
    return l1, l2


def _bessel_rbf(dist):
    n = jnp.arange(1, NUM_BASIS + 1, dtype=jnp.float32)
    return jnp.sqrt(2.0 / CUTOFF) * jnp.sin(n * jnp.pi * dist / CUTOFF) / dist


def _cosine_cutoff(dist):
    return 0.5 * (jnp.cos(jnp.pi * dist / CUTOFF) + 1.0) * (dist < CUTOFF).astype(jnp.float32)


def kernel(at_no, coord, edge_index, shifts, charge, spin, params):
    src = edge_index[0]
    dst = edge_index[1]
    N = coord.shape[0]
    vec = coord[src] - coord[dst] - shifts
    x = params['embed_table'][at_no]
    x_scalar0 = x @ params['node_lin_W'] + params['node_lin_b']

    def energy_fn(vec):
        dist = jnp.sqrt(jnp.sum(vec * vec, axis=-1, keepdims=True) + 1e-10)
        rbf = _bessel_rbf(dist)
        fcut = _cosine_cutoff(dist)
        u = vec[:, jnp.array([1, 2, 0])] / dist
        l1, l2 = _sph_harm(u)
        xs = x_scalar0
        xv0 = jnp.zeros((N, 128), jnp.float32)
        xv1 = jnp.zeros((N, 3, 64), jnp.float32)
        xv2 = jnp.zeros((N, 5, 32), jnp.float32)
        for b in range(N_BLOCKS):
            phi = _silu(xs @ params['msg_W1'][b] + params['msg_b1'][b])
            phi = phi @ params['msg_W2'][b] + params['msg_b2'][b]
            W = (rbf @ params['msg_Wrbf'][b] + params['msg_brbf'][b]) * fcut
            cat = jnp.concatenate([
                phi, xv0, xv1.reshape(N, 192), xv2.reshape(N, 160)], axis=1)
            ce = cat[dst]
            pr = ce[:, :576] * W
            ds = pr[:, :128]
            g1 = pr[:, 128:352]
            g2 = pr[:, 352:576]
            xv0d = ce[:, 576:704]
            xv1d = ce[:, 704:896].reshape(-1, 3, 64)
            xv2d = ce[:, 896:1056].reshape(-1, 5, 32)
            dv0 = xv0d * g1[:, :128] + g2[:, :128]
            dv1 = xv1d * g1[:, None, 128:192] + l1[:, :, None] * g2[:, None, 128:192]
            dv2 = xv2d * g1[:, None, 192:224] + l2[:, :, None] * g2[:, None, 192:224]
            upd = jnp.concatenate([
                ds, dv0, dv1.reshape(-1, 192), dv2.reshape(-1, 160)], axis=1)
            agg = jax.ops.segment_sum(upd, src, num_segments=N)
            xs = xs + agg[:, :128]
            xv0 = xv0 + agg[:, 128:256]
            xv1 = xv1 + agg[:, 256:448].reshape(N, 3, 64)
            xv2 = xv2 + agg[:, 448:608].reshape(N, 5, 32)
            U0 = xv0 @ params['upd_U0'][b]
            V0 = xv0 @ params['upd_V0'][b]
            U1 = xv1 @ params['upd_U1'][b]
            V1 = xv1 @ params['upd_V1'][b]
            U2 = xv2 @ params['upd_U2'][b]
            V2 = xv2 @ params['upd_V2'][b]
            norms = jnp.concatenate([
                jnp.sqrt(V0 * V0 + 1e-10),
                jnp.sqrt(jnp.sum(V1 * V1, axis=1) + 1e-10),
                jnp.sqrt(jnp.sum(V2 * V2, axis=1) + 1e-10)], axis=-1)
            h = _silu(jnp.concatenate([xs, norms], axis=-1) @ params['upd_W1'][b] + params['upd_b1'][b])
            h = h @ params['upd_W2'][b] + params['upd_b2'][b]
            a_ss = h[:, :NODE_DIM]
            a_sv = h[:, NODE_DIM:2 * NODE_DIM]
            a_vv = h[:, 2 * NODE_DIM:]
            dot = jnp.concatenate([
                U0 * V0, jnp.sum(U1 * V1, axis=1), jnp.sum(U2 * V2, axis=1)], axis=-1)
            xs = xs + a_ss + a_sv * (dot @ params['upd_Wdot'][b])
            xv0 = xv0 + a_vv[:, :128] * U0
            xv1 = xv1 + a_vv[:, None, 128:192] * U1
            xv2 = xv2 + a_vv[:, None, 192:224] * U2
        hh = _silu(xs @ params['out_W1'] + params['out_b1'])
        energies = (hh @ params['out_W2'] + params['out_b2']).reshape(-1)
        return jnp.sum(energies), energies

    (etot, energies_raw), edge_grad = jax.value_and_grad(energy_fn, has_aux=True)(vec)
    energies = energies_raw + params['atom_sp'][at_no]
    energy = jnp.sum(energies)

    acc = _force_virial(edge_grad, vec, src, dst)
    forces = acc[:, :3]
    virials = acc[:, 3:12].reshape(N, 3, 3)
    virial = jnp.sum(virials, axis=0)
    virial = 0.5 * (virial + virial.T)
    return energy, energies, forces, virial, virials
